# baseline (speedup 1.0000x reference)
import jax
import jax.numpy as jnp
from jax import lax
from jax.experimental import pallas as pl
from jax.experimental.pallas import tpu as pltpu, tpu_sc as plsc

NC = 2
NS = 16
NW = NC * NS
D = 128
T = 20
BBLK = 128
NBUF = 6


def _gather_kernel(idx_hbm, table_hbm, out_hbm, idx_v, bufs, gsem, wsem):
    wid = lax.axis_index("s") * NC + lax.axis_index("c")
    base = wid * BBLK

    pltpu.sync_copy(idx_hbm.at[wid], idx_v)

    def wait_gather():
        pltpu.make_async_copy(
            table_hbm.at[pl.ds(0, BBLK)], bufs.at[0], gsem).wait()

    def wait_write():
        pltpu.make_async_copy(
            bufs.at[0], out_hbm.at[0, pl.ds(base, BBLK)], wsem).wait()

    for t in range(NBUF - 1):
        pltpu.async_copy(table_hbm.at[idx_v.at[t]], bufs.at[t], gsem)

    def body(t, _):
        nt = t + NBUF - 1

        @pl.when(jnp.logical_and(nt < T, t >= 1))
        def _():
            wait_write()

        @pl.when(nt < T)
        def _():
            pltpu.async_copy(
                table_hbm.at[idx_v.at[nt]], bufs.at[lax.rem(nt, NBUF)], gsem)

        wait_gather()
        pltpu.async_copy(
            bufs.at[lax.rem(t, NBUF)], out_hbm.at[t, pl.ds(base, BBLK)], wsem)
        return ()

    lax.fori_loop(0, T, body, (), unroll=False)

    for _ in range(NBUF):
        wait_write()


@jax.jit
def _embedding_lookup(idx3, table):
    b_rows = idx3.shape[0] * idx3.shape[2]
    mesh = plsc.VectorSubcoreMesh(core_axis_name="c", subcore_axis_name="s")
    out = pl.kernel(
        _gather_kernel,
        out_type=jax.ShapeDtypeStruct((T, b_rows, D), jnp.float32),
        mesh=mesh,
        scratch_types=[
            pltpu.VMEM((T, BBLK), jnp.int32),
            pltpu.VMEM((NBUF, BBLK, D), jnp.float32),
            pltpu.SemaphoreType.DMA,
            pltpu.SemaphoreType.DMA,
        ],
        compiler_params=pltpu.CompilerParams(
            use_tc_tiling_on_sc=True, needs_layout_passes=True),
    )(idx3, table)
    return jnp.transpose(out, (1, 0, 2))


def kernel(batch_original_ids, embedding_weight):
    b, t = batch_original_ids.shape
    idx3 = batch_original_ids.reshape(NW, b // NW, t).transpose(0, 2, 1)
    return _embedding_lookup(idx3, embedding_weight)


# speedup vs baseline: 1.0013x; 1.0013x over previous
import jax
import jax.numpy as jnp
from jax import lax
from jax.experimental import pallas as pl
from jax.experimental.pallas import tpu as pltpu, tpu_sc as plsc

NC = 2
NS = 16
NW = NC * NS
D = 128
T = 20
BBLK = 128
NBUF = 6


def _gather_kernel(idx_hbm, table_hbm, out_hbm, idx_v, bufs, gsem, wsem):
    wid = lax.axis_index("s") * NC + lax.axis_index("c")
    base = wid * BBLK

    pltpu.sync_copy(idx_hbm.at[wid], idx_v)

    def wait_gather():
        pltpu.make_async_copy(
            table_hbm.at[pl.ds(0, BBLK)], bufs.at[0], gsem).wait()

    def wait_write():
        pltpu.make_async_copy(
            bufs.at[0], out_hbm.at[0, pl.ds(base, BBLK)], wsem).wait()

    for t in range(NBUF - 1):
        pltpu.async_copy(table_hbm.at[idx_v.at[t]], bufs.at[t], gsem)

    def body(t, _):
        nt = t + NBUF - 1

        @pl.when(jnp.logical_and(nt < T, t >= 1))
        def _():
            wait_write()

        @pl.when(nt < T)
        def _():
            pltpu.async_copy(
                table_hbm.at[idx_v.at[nt]], bufs.at[lax.rem(nt, NBUF)], gsem)

        wait_gather()
        pltpu.async_copy(
            bufs.at[lax.rem(t, NBUF)], out_hbm.at[t, pl.ds(base, BBLK)], wsem)
        return ()

    lax.fori_loop(0, T, body, (), unroll=False)

    for _ in range(NBUF):
        wait_write()


@jax.jit
def _embedding_lookup(idx3, table):
    b_rows = idx3.shape[0] * idx3.shape[2]
    mesh = plsc.VectorSubcoreMesh(core_axis_name="c", subcore_axis_name="s")
    out = pl.kernel(
        _gather_kernel,
        out_type=jax.ShapeDtypeStruct((T, b_rows, D), jnp.float32),
        mesh=mesh,
        scratch_types=[
            pltpu.VMEM((T, BBLK), jnp.int32),
            pltpu.VMEM((NBUF, BBLK, D), jnp.float32),
            pltpu.SemaphoreType.DMA,
            pltpu.SemaphoreType.DMA,
        ],
        compiler_params=pltpu.CompilerParams(
            use_tc_tiling_on_sc=True, needs_layout_passes=True),
    )(idx3, table)
    return jnp.transpose(out, (1, 0, 2))


def kernel(batch_original_ids, embedding_weight):
    b, t = batch_original_ids.shape
    idx3 = batch_original_ids.T.reshape(t, NW, b // NW).transpose(1, 0, 2)
    return _embedding_lookup(idx3, embedding_weight)
